# baseline (speedup 1.0000x reference)
import jax
import jax.numpy as jnp
from jax.experimental import pallas as pl

ALPHA = 0.2
D = 64
HW = 128
BLK = 2048


def _elu(v):
    return jnp.where(v > 0, v, jnp.exp(jnp.minimum(v, 0.0)) - 1.0)


def _lrelu(v):
    return jnp.where(v > 0, v, ALPHA * v)


def _cdiv(a, b):
    return (a + b - 1) // b


def _bs(w):
    return pl.BlockSpec((BLK, w), lambda i: (i, 0))


def _hf(x, W, a):
    h = jnp.dot(x, W, preferred_element_type=jnp.float32)
    f1 = jnp.sum(h * a[:, :D], axis=1, keepdims=True)
    f2 = jnp.sum(h * a[:, D:], axis=1, keepdims=True)
    z = jnp.zeros((h.shape[0], HW - D - 2), jnp.float32)
    return jnp.concatenate([h, f1, f2, z], axis=1)


def _wself(hf):
    return jnp.exp(-_lrelu(hf[:, D:D + 1] + hf[:, D + 1:D + 2]))


def _head_out(hf, acc):
    num = _wself(hf) * hf[:, :D] + acc[:, :D]
    den = _wself(hf) + acc[:, D:D + 1]
    return _elu(num / den)


def _prep2_body(x_ref, w1_ref, a1_ref, w2_ref, a2_ref, o1_ref, o2_ref):
    x = x_ref[...]
    o1_ref[...] = _hf(x, w1_ref[...], a1_ref[...])
    o2_ref[...] = _hf(x, w2_ref[...], a2_ref[...])


def _tc_prep2(x, W1, a1, W2, a2):
    n, fin = x.shape
    grid = _cdiv(n, BLK)
    return pl.pallas_call(
        _prep2_body,
        grid=(grid,),
        in_specs=[_bs(fin),
                  pl.BlockSpec((fin, D), lambda i: (0, 0)),
                  pl.BlockSpec((1, 2 * D), lambda i: (0, 0)),
                  pl.BlockSpec((fin, D), lambda i: (0, 0)),
                  pl.BlockSpec((1, 2 * D), lambda i: (0, 0))],
        out_specs=[_bs(HW), _bs(HW)],
        out_shape=[jax.ShapeDtypeStruct((n, HW), jnp.float32),
                   jax.ShapeDtypeStruct((n, HW), jnp.float32)],
    )(x, W1, a1, W2, a2)


def _epi2_prep1_body(hf1_ref, acc1_ref, hf2_ref, acc2_ref, w_ref, a_ref,
                     o_ref):
    xcat = jnp.concatenate([_head_out(hf1_ref[...], acc1_ref[...]),
                            _head_out(hf2_ref[...], acc2_ref[...])], axis=1)
    o_ref[...] = _hf(xcat, w_ref[...], a_ref[...])


def _tc_epi2_prep1(hf1, acc1, hf2, acc2, W, a):
    n = hf1.shape[0]
    grid = _cdiv(n, BLK)
    return pl.pallas_call(
        _epi2_prep1_body,
        grid=(grid,),
        in_specs=[_bs(HW), _bs(D + 1), _bs(HW), _bs(D + 1),
                  pl.BlockSpec((2 * D, D), lambda i: (0, 0)),
                  pl.BlockSpec((1, 2 * D), lambda i: (0, 0))],
        out_specs=_bs(HW),
        out_shape=jax.ShapeDtypeStruct((n, HW), jnp.float32),
    )(hf1, acc1, hf2, acc2, W, a)


def _epi1_prep2_body(hf_ref, acc_ref, w1_ref, a1_ref, w2_ref, a2_ref,
                     o1_ref, o2_ref):
    x = _head_out(hf_ref[...], acc_ref[...])
    o1_ref[...] = _hf(x, w1_ref[...], a1_ref[...])
    o2_ref[...] = _hf(x, w2_ref[...], a2_ref[...])


def _tc_epi1_prep2(hf, acc, W1, a1, W2, a2):
    n = hf.shape[0]
    grid = _cdiv(n, BLK)
    return pl.pallas_call(
        _epi1_prep2_body,
        grid=(grid,),
        in_specs=[_bs(HW), _bs(D + 1),
                  pl.BlockSpec((D, D), lambda i: (0, 0)),
                  pl.BlockSpec((1, 2 * D), lambda i: (0, 0)),
                  pl.BlockSpec((D, D), lambda i: (0, 0)),
                  pl.BlockSpec((1, 2 * D), lambda i: (0, 0))],
        out_specs=[_bs(HW), _bs(HW)],
        out_shape=[jax.ShapeDtypeStruct((n, HW), jnp.float32),
                   jax.ShapeDtypeStruct((n, HW), jnp.float32)],
    )(hf, acc, W1, a1, W2, a2)


def _epi1_body(hf_ref, acc_ref, o_ref):
    o_ref[...] = _head_out(hf_ref[...], acc_ref[...])


def _tc_epi1(hf, acc):
    n = hf.shape[0]
    grid = _cdiv(n, BLK)
    return pl.pallas_call(
        _epi1_body,
        grid=(grid,),
        in_specs=[_bs(HW), _bs(D + 1)],
        out_specs=_bs(D),
        out_shape=jax.ShapeDtypeStruct((n, D), jnp.float32),
    )(hf, acc)


def _final_body(hf_ref, acc_ref, w_ref, b_ref, o_ref):
    th = _head_out(hf_ref[...], acc_ref[...])
    logits = jnp.sum(th * w_ref[...], axis=1, keepdims=True) + b_ref[0, 0]
    o_ref[...] = 1.0 / (1.0 + jnp.exp(-logits))


def _tc_final(hf, acc, W_out, b_out):
    n = hf.shape[0]
    return pl.pallas_call(
        _final_body,
        grid=(1,),
        in_specs=[pl.BlockSpec((n, HW), lambda i: (0, 0)),
                  pl.BlockSpec((n, D + 1), lambda i: (0, 0)),
                  pl.BlockSpec((1, D), lambda i: (0, 0)),
                  pl.BlockSpec((1, 1), lambda i: (0, 0))],
        out_specs=pl.BlockSpec((n, 1), lambda i: (0, 0)),
        out_shape=jax.ShapeDtypeStruct((n, 1), jnp.float32),
    )(hf, acc, W_out.reshape(1, D), b_out.reshape(1, 1))


def _edge_acc(hf, srcs, dsts, n):
    w = jnp.exp(-_lrelu(hf[srcs, D] + hf[dsts, D + 1]))
    payload = jnp.concatenate([w[:, None] * hf[dsts, :D], w[:, None]], axis=1)
    return jax.ops.segment_sum(payload, srcs, num_segments=n)


def kernel(repo, repo_users, users, user_edges, teams, team_users, params):
    n_users = users.shape[0]
    n_repo = n_users + 1
    p = params

    x = jnp.concatenate([users, repo[None, :]], axis=0)
    srcs_r = repo_users.astype(jnp.int32)
    dsts_r = jnp.full_like(srcs_r, n_repo - 1)
    hf1, hf2 = _tc_prep2(x, p['W_repo_0'], p['a_repo_0'],
                         p['W_repo_1'], p['a_repo_1'])
    acc1 = _edge_acc(hf1, srcs_r, dsts_r, n_repo)
    acc2 = _edge_acc(hf2, srcs_r, dsts_r, n_repo)
    hfo = _tc_epi2_prep1(hf1, acc1, hf2, acc2,
                         p['W_repo_out'], p['a_repo_out'])
    acco = _edge_acc(hfo, srcs_r, dsts_r, n_repo)

    e_rand = user_edges.shape[1] - n_users
    srcs_u = user_edges[0, :e_rand].astype(jnp.int32)
    dsts_u = user_edges[1, :e_rand].astype(jnp.int32)
    hf1, hf2 = _tc_epi1_prep2(hfo[:n_users], acco[:n_users],
                              p['W_user_0'], p['a_user_0'],
                              p['W_user_1'], p['a_user_1'])
    acc1 = _edge_acc(hf1, srcs_u, dsts_u, n_users)
    acc2 = _edge_acc(hf2, srcs_u, dsts_u, n_users)
    hfo = _tc_epi2_prep1(hf1, acc1, hf2, acc2,
                         p['W_user_out'], p['a_user_out'])
    acco = _edge_acc(hfo, srcs_u, dsts_u, n_users)
    user_h = _tc_epi1(hfo, acco)

    t_total = team_users.shape[0]
    nt = n_users + t_total
    x2 = jnp.concatenate([user_h, teams], axis=0)
    srcs_t = jnp.repeat(jnp.arange(t_total, dtype=jnp.int32) + n_users,
                        team_users.shape[1])
    dsts_t = team_users.reshape(-1).astype(jnp.int32)
    hf1, hf2 = _tc_prep2(x2, p['W_team_0'], p['a_team_0'],
                         p['W_team_1'], p['a_team_1'])
    acc1 = _edge_acc(hf1, srcs_t, dsts_t, nt)
    acc2 = _edge_acc(hf2, srcs_t, dsts_t, nt)
    hfo = _tc_epi2_prep1(hf1, acc1, hf2, acc2,
                         p['W_team_out'], p['a_team_out'])
    acco_t = _edge_acc(hfo, srcs_t, dsts_t, nt)[n_users:]
    return _tc_final(hfo[n_users:], acco_t, p['W_out'], p['b_out'])
